# Initial kernel scaffold; baseline (speedup 1.0000x reference)
import jax
import jax.numpy as jnp
from jax.experimental import pallas as pl


def kernel(queries, truths, data):
    raise NotImplementedError("write your pallas kernel here")


# speedup vs baseline: 1.8758x; 1.8758x over previous
import functools

import jax
import jax.numpy as jnp
from jax.experimental import pallas as pl
from jax.experimental.pallas import tpu as pltpu

_EPS = 1e-12
_ATOL = 1e-8
_RTOL = 1e-5


def _rank_kernel(q_ref, t_ref, d_ref, out_ref, qn_ref, te_ref):
    k = pl.program_id(0)

    @pl.when(k == 0)
    def _init():
        q = q_ref[...]
        t = t_ref[...]
        qn = q / jnp.maximum(
            jnp.sqrt(jnp.sum(q * q, axis=1, keepdims=True)), _EPS)
        tn = t / jnp.maximum(
            jnp.sqrt(jnp.sum(t * t, axis=1, keepdims=True)), _EPS)
        qn_ref[...] = qn
        m = jax.lax.dot_general(qn, tn, (((1,), (1,)), ((), ())))
        nq = m.shape[0]
        eye = (jax.lax.broadcasted_iota(jnp.int32, (nq, nq), 0)
               == jax.lax.broadcasted_iota(jnp.int32, (nq, nq), 1))
        thr = jnp.sum(jnp.where(eye, m, 0.0), axis=0, keepdims=True)
        te_ref[...] = thr - (_ATOL + _RTOL * jnp.abs(thr))

    d = d_ref[...]
    s = jax.lax.dot_general(d, qn_ref[...], (((1,), (1,)), ((), ())))
    norm = jnp.maximum(
        jnp.sqrt(jnp.sum(d * d, axis=1, keepdims=True)), _EPS)
    ge = s >= te_ref[...] * norm
    cnt = jnp.sum(ge.astype(jnp.int32), axis=0, keepdims=True)

    @pl.when(k == 0)
    def _first():
        out_ref[...] = cnt - 1

    @pl.when(k != 0)
    def _rest():
        out_ref[...] = out_ref[...] + cnt


@functools.partial(jax.jit, static_argnames=("block",))
def _rank(queries, truths, data, block=512):
    n, d = data.shape
    nq = queries.shape[0]
    grid = (n // block,)
    return pl.pallas_call(
        _rank_kernel,
        grid=grid,
        in_specs=[
            pl.BlockSpec((nq, d), lambda k: (0, 0)),
            pl.BlockSpec((nq, d), lambda k: (0, 0)),
            pl.BlockSpec((block, d), lambda k: (k, 0)),
        ],
        out_specs=pl.BlockSpec((1, nq), lambda k: (0, 0)),
        out_shape=jax.ShapeDtypeStruct((1, nq), jnp.int32),
        scratch_shapes=[
            pltpu.VMEM((nq, d), jnp.float32),
            pltpu.VMEM((1, nq), jnp.float32),
        ],
        compiler_params=pltpu.CompilerParams(
            dimension_semantics=("arbitrary",),
        ),
    )(queries, truths, data)


def kernel(queries, truths, data):
    return _rank(queries, truths, data)
